# baseline (speedup 1.0000x reference)
import functools

import jax
import jax.numpy as jnp
from jax import lax
from jax.experimental import pallas as pl
from jax.experimental.pallas import tpu as pltpu
from jax.experimental.pallas import tpu_sc as plsc

F32 = jnp.float32
I32 = jnp.int32

_NC, _NS, _NW = 2, 16, 32


def _mesh():
    return plsc.VectorSubcoreMesh(core_axis_name="c", subcore_axis_name="s")


_SC_PARAMS = pltpu.CompilerParams(needs_layout_passes=False)


def _wid():
    return lax.axis_index("s") * _NC + lax.axis_index("c")


def _xproj_body(x_ref, w_ref, o_ref):
    o_ref[...] = jnp.dot(x_ref[...], w_ref[...], preferred_element_type=F32)


def _xproj(x, wcat):
    n = x.shape[0]
    blk = 2000
    return pl.pallas_call(
        _xproj_body,
        grid=(n // blk,),
        in_specs=[
            pl.BlockSpec((blk, 128), lambda i: (i, 0)),
            pl.BlockSpec((128, 128), lambda i: (0, 0)),
        ],
        out_specs=pl.BlockSpec((blk, 128), lambda i: (i, 0)),
        out_shape=jax.ShapeDtypeStruct((n, 128), F32),
    )(x, wcat)


def _eaw_body(e1_ref, e2_ref, we_ref, bm_ref, o_ref):
    o_ref[:, :64] = (jnp.dot(e1_ref[...], we_ref[...], preferred_element_type=F32)
                     + bm_ref[...])
    o_ref[:, 64:] = (jnp.dot(e2_ref[...], we_ref[...], preferred_element_type=F32)
                     + bm_ref[...])


def _eaw(edge_attr, we, b_msg):
    e = edge_attr.shape[0]
    e2 = e // 2
    blk = 1000
    grid = e2 // blk
    return pl.pallas_call(
        _eaw_body,
        grid=(grid,),
        in_specs=[
            pl.BlockSpec((blk, 64), lambda i: (i, 0)),
            pl.BlockSpec((blk, 64), lambda i, g=grid: (i + g, 0)),
            pl.BlockSpec((64, 64), lambda i: (0, 0)),
            pl.BlockSpec((1, 64), lambda i: (0, 0)),
        ],
        out_specs=pl.BlockSpec((blk, 128), lambda i: (i, 0)),
        out_shape=jax.ShapeDtypeStruct((e2, 128), F32),
    )(edge_attr, edge_attr, we, b_msg.reshape(1, 64))


def _make_scgather(e):
    e2 = e // 2
    bc = 128
    nbc = e2 // bc
    per = nbc // _NW
    extra = nbc - per * _NW

    @functools.partial(
        pl.kernel,
        out_type=jax.ShapeDtypeStruct((e2, 128), F32),
        mesh=_mesh(),
        compiler_params=_SC_PARAMS,
        scratch_types=[
            pltpu.VMEM((bc,), I32),
            pltpu.VMEM((bc,), I32),
            pltpu.VMEM((bc,), I32),
            pltpu.VMEM((bc,), I32),
            pltpu.VMEM((bc, 128), F32),
            pltpu.VMEM((bc, 128), F32),
            pltpu.VMEM((bc, 128), F32),
            pltpu.VMEM((bc, 128), F32),
            pltpu.VMEM((bc, 128), F32),
            pltpu.SemaphoreType.DMA,
        ],
    )
    def scgather(src_hbm, dst_hbm, xsd_hbm, m0_hbm,
                 s1i, d1i, s2i, d2i, bs1, bd1, bs2, bd2, bm, sem):
        w = _wid()
        nb = jnp.where(w < extra, per + 1, per)

        def body(t, carry):
            off = pl.multiple_of((w + _NW * t) * bc, 8)
            pltpu.sync_copy(src_hbm.at[pl.ds(off, bc)], s1i)
            pltpu.sync_copy(dst_hbm.at[pl.ds(off, bc)], d1i)
            pltpu.sync_copy(src_hbm.at[pl.ds(e2 + off, bc)], s2i)
            pltpu.sync_copy(dst_hbm.at[pl.ds(e2 + off, bc)], d2i)
            cps = [
                pltpu.async_copy(xsd_hbm.at[s1i], bs1, sem),
                pltpu.async_copy(xsd_hbm.at[d1i], bd1, sem),
                pltpu.async_copy(xsd_hbm.at[s2i], bs2, sem),
                pltpu.async_copy(xsd_hbm.at[d2i], bd2, sem),
            ]
            for cp in cps:
                cp.wait()

            def row(r, c2):
                for cc in range(4):
                    lo = pl.ds(cc * 16, 16)
                    hi = pl.ds(64 + cc * 16, 16)
                    bm[r, lo] = bs1[r, lo] + bd1[r, hi]
                    bm[r, hi] = bs2[r, lo] + bd2[r, hi]
                return c2
            lax.fori_loop(0, bc, row, 0)

            pltpu.sync_copy(bm, m0_hbm.at[pl.ds(off, bc)])
            return carry

        lax.fori_loop(0, nb, body, 0)

    return scgather


def _mlp_body(m0_ref, ew_ref, w1_ref, b1_ref, w2_ref, b2_ref, h_ref):
    m = m0_ref[...] + ew_ref[...]
    t = jnp.maximum(jnp.dot(m, w1_ref[...], preferred_element_type=F32)
                    + b1_ref[...], 0.0)
    h_ref[...] = jnp.dot(t, w2_ref[...], preferred_element_type=F32) + b2_ref[...]


def _mlp(m0h, eawh, w1, b1, w2, b2):
    e2 = m0h.shape[0]
    blk = 1000
    zero = jnp.zeros((64, 64), F32)
    w1d = jnp.block([[w1, zero], [zero, w1]])
    w2d = jnp.block([[w2, zero], [zero, w2]])
    b1d = jnp.concatenate([b1, b1]).reshape(1, 128)
    b2d = jnp.concatenate([b2, b2]).reshape(1, 128)
    row = lambda i: (i, 0)
    whole = lambda i: (0, 0)
    return pl.pallas_call(
        _mlp_body,
        grid=(e2 // blk,),
        in_specs=[
            pl.BlockSpec((blk, 128), row),
            pl.BlockSpec((blk, 128), row),
            pl.BlockSpec((128, 128), whole),
            pl.BlockSpec((1, 128), whole),
            pl.BlockSpec((128, 128), whole),
            pl.BlockSpec((1, 128), whole),
        ],
        out_specs=pl.BlockSpec((blk, 128), row),
        out_shape=jax.ShapeDtypeStruct((e2, 128), F32),
    )(m0h, eawh, w1d, b1d, w2d, b2d)


def _make_scsegmax(e, n):
    e2 = e // 2
    rng = (n + _NW - 1) // _NW
    lastr = n - (_NW - 1) * rng
    chunk = 4000
    nchunks = e // chunk
    cap = chunk + 224

    @functools.partial(
        pl.kernel,
        out_type=jax.ShapeDtypeStruct((n * 64,), F32),
        mesh=_mesh(),
        compiler_params=_SC_PARAMS,
        scratch_types=[
            pltpu.VMEM((rng * 64,), F32),
            pltpu.VMEM((chunk,), I32),
            pltpu.VMEM((cap,), I32),
            pltpu.VMEM((cap,), I32),
            pltpu.VMEM((128, 128), F32),
            pltpu.SemaphoreType.DMA,
        ],
    )
    def scsegmax(dst_hbm, h_hbm, agg_hbm, aggt, dbuf, capd, capg, hbuf, sem):
        w = _wid()
        base = w * rng
        ninf = jnp.full((16,), -jnp.inf, F32)

        def initb(k, c):
            aggt[pl.ds(k * 16, 16)] = ninf
            return c
        lax.fori_loop(0, rng * 4, initb, 0)

        def do_chunk(cidx, carry):
            off = pl.multiple_of(cidx * chunk, 8)
            pltpu.sync_copy(dst_hbm.at[pl.ds(off, chunk)], dbuf)

            iota16 = lax.iota(I32, 16)

            trash = chunk + 208 + iota16

            def scan(i, cnt):
                d16 = dbuf[pl.ds(i * 16, 16)]
                t = d16 - base
                m = (t >= 0) & (t < rng)
                ev = off + i * 16 + iota16
                half = ev >= e2
                vv = t + jnp.where(half, 32768, 0)
                gv = ev - jnp.where(half, e2, 0)
                cs = plsc.cumsum(jnp.where(m, 1, 0).astype(I32))
                pos = jnp.where(m, cnt + cs - 1, trash)
                plsc.store_scatter(capd, [pos], vv)
                plsc.store_scatter(capg, [pos], gv)
                return cnt + cs[15]

            cnt = lax.fori_loop(0, chunk // 16, scan, jnp.int32(0))

            zz = jnp.zeros((16,), I32)

            def padb(j, c):
                capg[pl.ds(cnt + j * 16, 16)] = zz
                return c
            lax.fori_loop(0, 8, padb, 0)

            nslab = (cnt + 127) // 128

            def slab(s, c):
                sl = pl.ds(pl.multiple_of(s * 128, 8), 128)
                pltpu.async_copy(h_hbm.at[capg.at[sl]], hbuf, sem).wait()
                lo = s * 128
                hi = jnp.minimum(cnt - lo, 128)

                def acc(j, c2):
                    vv = capd[pl.ds(lo + j, 16)][0]
                    d = vv & 32767
                    hb = (vv >> 15) * 64
                    rb = pl.multiple_of(d * 64, 8)
                    for cc in range(4):
                        hv = hbuf[j, pl.ds(hb + cc * 16, 16)]
                        av = aggt[pl.ds(rb + cc * 16, 16)]
                        aggt[pl.ds(rb + cc * 16, 16)] = jnp.maximum(av, hv)
                    return c2

                lax.fori_loop(0, hi, acc, 0)
                return c

            lax.fori_loop(0, nslab, slab, 0)
            return carry

        lax.fori_loop(0, nchunks, do_chunk, 0)

        def fixb(k, c):
            v = aggt[pl.ds(k * 16, 16)]
            aggt[pl.ds(k * 16, 16)] = jnp.where(jnp.abs(v) < jnp.inf, v, 0.0)
            return c
        lax.fori_loop(0, rng * 4, fixb, 0)

        @pl.when(w < _NW - 1)
        def _():
            pltpu.sync_copy(aggt.at[pl.ds(0, rng * 64)],
                            agg_hbm.at[pl.ds(base * 64, rng * 64)])

        @pl.when(w == _NW - 1)
        def _():
            pltpu.sync_copy(aggt.at[pl.ds(0, lastr * 64)],
                            agg_hbm.at[pl.ds(base * 64, lastr * 64)])

    return scsegmax


def _upd_body(x_ref, ag_ref, wox_ref, woa_ref, bo_ref, o_ref):
    o_ref[...] = (jnp.dot(x_ref[...], wox_ref[...], preferred_element_type=F32)
                  + jnp.dot(ag_ref[...], woa_ref[...], preferred_element_type=F32)
                  + bo_ref[...])


def _update(x, agg, wox, woa, b_out):
    n = x.shape[0]
    blk = 2000
    return pl.pallas_call(
        _upd_body,
        grid=(n // blk,),
        in_specs=[
            pl.BlockSpec((blk, 128), lambda i: (i, 0)),
            pl.BlockSpec((blk, 64), lambda i: (i, 0)),
            pl.BlockSpec((128, 64), lambda i: (0, 0)),
            pl.BlockSpec((64, 64), lambda i: (0, 0)),
            pl.BlockSpec((1, 64), lambda i: (0, 0)),
        ],
        out_specs=pl.BlockSpec((blk, 64), lambda i: (i, 0)),
        out_shape=jax.ShapeDtypeStruct((n, 64), F32),
    )(x, agg, wox, woa, b_out.reshape(1, 64))


@jax.jit
def kernel(x, adj, edge_attr, W_msg, b_msg, W1, b1, W2, b2, W_out, b_out):
    n = x.shape[0]
    e = adj.shape[1]
    src = adj[0].astype(I32)
    dst = adj[1].astype(I32)

    xsd = _xproj(x, jnp.concatenate([W_msg[:128], W_msg[128:256]], axis=1))
    eawh = _eaw(edge_attr, W_msg[256:], b_msg)

    m0h = _make_scgather(e)(src, dst, xsd)

    h2 = _mlp(m0h, eawh, W1, b1, W2, b2)

    agg = _make_scsegmax(e, n)(dst, h2).reshape(n, 64)

    return _update(x, agg, W_out[:128], W_out[128:], b_out)


# speedup vs baseline: 1.7924x; 1.7924x over previous
import functools

import jax
import jax.numpy as jnp
from jax import lax
from jax.experimental import pallas as pl
from jax.experimental.pallas import tpu as pltpu
from jax.experimental.pallas import tpu_sc as plsc

F32 = jnp.float32
I32 = jnp.int32

_NC, _NS, _NW = 2, 16, 32


def _mesh():
    return plsc.VectorSubcoreMesh(core_axis_name="c", subcore_axis_name="s")


_SC_PARAMS = pltpu.CompilerParams(needs_layout_passes=False)


def _wid():
    return lax.axis_index("s") * _NC + lax.axis_index("c")


def _xproj_body(x_ref, w_ref, o_ref):
    o_ref[...] = jnp.dot(x_ref[...], w_ref[...], preferred_element_type=F32)


def _xproj(x, wcat):
    n = x.shape[0]
    blk = 2000
    return pl.pallas_call(
        _xproj_body,
        grid=(n // blk,),
        in_specs=[
            pl.BlockSpec((blk, 128), lambda i: (i, 0)),
            pl.BlockSpec((128, 128), lambda i: (0, 0)),
        ],
        out_specs=pl.BlockSpec((blk, 128), lambda i: (i, 0)),
        out_shape=jax.ShapeDtypeStruct((n, 128), F32),
    )(x, wcat)


def _eaw_body(e1_ref, e2_ref, we_ref, bm_ref, o_ref):
    o_ref[:, :64] = (jnp.dot(e1_ref[...], we_ref[...], preferred_element_type=F32)
                     + bm_ref[...])
    o_ref[:, 64:] = (jnp.dot(e2_ref[...], we_ref[...], preferred_element_type=F32)
                     + bm_ref[...])


def _eaw(edge_attr, we, b_msg):
    e = edge_attr.shape[0]
    e2 = e // 2
    blk = 1000
    grid = e2 // blk
    return pl.pallas_call(
        _eaw_body,
        grid=(grid,),
        in_specs=[
            pl.BlockSpec((blk, 64), lambda i: (i, 0)),
            pl.BlockSpec((blk, 64), lambda i, g=grid: (i + g, 0)),
            pl.BlockSpec((64, 64), lambda i: (0, 0)),
            pl.BlockSpec((1, 64), lambda i: (0, 0)),
        ],
        out_specs=pl.BlockSpec((blk, 128), lambda i: (i, 0)),
        out_shape=jax.ShapeDtypeStruct((e2, 128), F32),
    )(edge_attr, edge_attr, we, b_msg.reshape(1, 64))


def _make_scgather(e):
    e2 = e // 2
    bc = 128
    nbc = e2 // bc
    per = nbc // _NW
    extra = nbc - per * _NW

    @functools.partial(
        pl.kernel,
        out_type=jax.ShapeDtypeStruct((e2, 128), F32),
        mesh=_mesh(),
        compiler_params=_SC_PARAMS,
        scratch_types=[
            pltpu.VMEM((bc,), I32),
            pltpu.VMEM((bc,), I32),
            pltpu.VMEM((bc,), I32),
            pltpu.VMEM((bc,), I32),
            pltpu.VMEM((bc, 128), F32),
            pltpu.VMEM((bc, 128), F32),
            pltpu.VMEM((bc, 128), F32),
            pltpu.VMEM((bc, 128), F32),
            pltpu.VMEM((bc, 128), F32),
            pltpu.SemaphoreType.DMA,
        ],
    )
    def scgather(src_hbm, dst_hbm, xsd_hbm, m0_hbm,
                 s1i, d1i, s2i, d2i, bs1, bd1, bs2, bd2, bm, sem):
        w = _wid()
        nb = jnp.where(w < extra, per + 1, per)

        def body(t, carry):
            off = pl.multiple_of((w + _NW * t) * bc, 8)
            pltpu.sync_copy(src_hbm.at[pl.ds(off, bc)], s1i)
            pltpu.sync_copy(dst_hbm.at[pl.ds(off, bc)], d1i)
            pltpu.sync_copy(src_hbm.at[pl.ds(e2 + off, bc)], s2i)
            pltpu.sync_copy(dst_hbm.at[pl.ds(e2 + off, bc)], d2i)
            cps = [
                pltpu.async_copy(xsd_hbm.at[s1i], bs1, sem),
                pltpu.async_copy(xsd_hbm.at[d1i], bd1, sem),
                pltpu.async_copy(xsd_hbm.at[s2i], bs2, sem),
                pltpu.async_copy(xsd_hbm.at[d2i], bd2, sem),
            ]
            for cp in cps:
                cp.wait()

            def row(r, c2):
                for cc in range(4):
                    lo = pl.ds(cc * 16, 16)
                    hi = pl.ds(64 + cc * 16, 16)
                    bm[r, lo] = bs1[r, lo] + bd1[r, hi]
                    bm[r, hi] = bs2[r, lo] + bd2[r, hi]
                return c2
            lax.fori_loop(0, bc, row, 0)

            pltpu.sync_copy(bm, m0_hbm.at[pl.ds(off, bc)])
            return carry

        lax.fori_loop(0, nb, body, 0)

    return scgather


def _mlp_body(m0_ref, ew_ref, w1_ref, b1_ref, w2_ref, b2_ref, h_ref):
    m = m0_ref[...] + ew_ref[...]
    t = jnp.maximum(jnp.dot(m, w1_ref[...], preferred_element_type=F32)
                    + b1_ref[...], 0.0)
    h_ref[...] = jnp.dot(t, w2_ref[...], preferred_element_type=F32) + b2_ref[...]


def _mlp(m0h, eawh, w1, b1, w2, b2):
    e2 = m0h.shape[0]
    blk = 1000
    zero = jnp.zeros((64, 64), F32)
    w1d = jnp.block([[w1, zero], [zero, w1]])
    w2d = jnp.block([[w2, zero], [zero, w2]])
    b1d = jnp.concatenate([b1, b1]).reshape(1, 128)
    b2d = jnp.concatenate([b2, b2]).reshape(1, 128)
    row = lambda i: (i, 0)
    whole = lambda i: (0, 0)
    return pl.pallas_call(
        _mlp_body,
        grid=(e2 // blk,),
        in_specs=[
            pl.BlockSpec((blk, 128), row),
            pl.BlockSpec((blk, 128), row),
            pl.BlockSpec((128, 128), whole),
            pl.BlockSpec((1, 128), whole),
            pl.BlockSpec((128, 128), whole),
            pl.BlockSpec((1, 128), whole),
        ],
        out_specs=pl.BlockSpec((blk, 128), row),
        out_shape=jax.ShapeDtypeStruct((e2, 128), F32),
    )(m0h, eawh, w1d, b1d, w2d, b2d)


def _make_scsegmax(e, n):
    e2 = e // 2
    rng = (n + _NW - 1) // _NW
    lastr = n - (_NW - 1) * rng
    chunk = 8000
    nchunks = e // chunk
    cap = chunk + 224

    @functools.partial(
        pl.kernel,
        out_type=jax.ShapeDtypeStruct((n * 64,), F32),
        mesh=_mesh(),
        compiler_params=_SC_PARAMS,
        scratch_types=[
            pltpu.VMEM(((rng + 1) * 64,), F32),
            pltpu.VMEM((chunk,), I32),
            pltpu.VMEM((cap,), I32),
            pltpu.VMEM((cap,), I32),
            pltpu.VMEM((128, 128), F32),
            pltpu.VMEM((128, 128), F32),
            pltpu.SemaphoreType.DMA,
            pltpu.SemaphoreType.DMA,
        ],
    )
    def scsegmax(dst_hbm, h_hbm, agg_hbm, aggt, dbuf, capd, capg,
                 hb0, hb1, sem0, sem1):
        w = _wid()
        base = w * rng
        ninf = jnp.full((16,), -jnp.inf, F32)

        def initb(k, c):
            aggt[pl.ds(k * 16, 16)] = ninf
            return c
        lax.fori_loop(0, (rng + 1) * 4, initb, 0)

        iota16 = lax.iota(I32, 16)
        trash = chunk + 208 + iota16
        ones = jnp.full((16,), 1, I32)
        zeros16 = jnp.zeros((16,), I32)

        def do_chunk(cidx, carry):
            off = pl.multiple_of(cidx * chunk, 8)
            pltpu.sync_copy(dst_hbm.at[pl.ds(off, chunk)], dbuf)

            def scan(i, cnt_v):
                d16 = dbuf[pl.ds(i * 16, 16)]
                t = d16 - base
                m = (t >= 0) & (t < rng)
                ev = off + i * 16 + iota16
                half = ev >= e2
                vv = t + jnp.where(half, 32768, 0)
                gv = ev - jnp.where(half, e2, 0)
                cs = plsc.cumsum(jnp.where(m, ones, zeros16))
                pos = jnp.where(m, cnt_v + cs - 1, trash)
                plsc.store_scatter(capd, [pos], vv)
                plsc.store_scatter(capg, [pos], gv)
                pc = plsc.all_reduce_population_count(m)
                return cnt_v + pc

            cnt_v = lax.fori_loop(0, chunk // 16, scan, zeros16)
            cnt = cnt_v[0]

            capd[pl.ds(cnt, 16)] = jnp.full((16,), rng, I32)
            def padb(j, c):
                capg[pl.ds(cnt + j * 16, 16)] = zeros16
                return c
            lax.fori_loop(0, 8, padb, 0)

            nslab = (cnt + 127) // 128

            def start(s, hb, sem):
                sl = pl.ds(pl.multiple_of(s * 128, 8), 128)
                return pltpu.async_copy(h_hbm.at[capg.at[sl]], hb, sem)

            def accum(s, hb):
                lo = s * 128
                hi = cnt - lo

                def group(g):
                    @pl.when(g * 16 < hi)
                    def _():
                        vv16 = capd[pl.ds(lo + g * 16, 16)]
                        for k in range(16):
                            vv = vv16[k]
                            d = vv & 32767
                            hb_col = (vv >> 15) * 64
                            rb = pl.multiple_of(d * 64, 8)
                            j = g * 16 + k
                            for cc in range(4):
                                hv = hb[j, pl.ds(hb_col + cc * 16, 16)]
                                av = aggt[pl.ds(rb + cc * 16, 16)]
                                aggt[pl.ds(rb + cc * 16, 16)] = jnp.maximum(av, hv)
                for g in range(8):
                    group(g)

            cp0 = start(0, hb0, sem0)

            def slab(s, c):
                even = lax.rem(s, 2) == 0

                @pl.when(even)
                def _():
                    pltpu.make_async_copy(h_hbm.at[capg.at[pl.ds(0, 128)]], hb0, sem0).wait()
                    @pl.when(s + 1 < nslab)
                    def _():
                        start(s + 1, hb1, sem1)
                    accum(s, hb0)

                @pl.when(jnp.logical_not(even))
                def _():
                    pltpu.make_async_copy(h_hbm.at[capg.at[pl.ds(0, 128)]], hb1, sem1).wait()
                    @pl.when(s + 1 < nslab)
                    def _():
                        start(s + 1, hb0, sem0)
                    accum(s, hb1)

                return c

            lax.fori_loop(0, nslab, slab, 0)
            return carry

        lax.fori_loop(0, nchunks, do_chunk, 0)

        def fixb(k, c):
            v = aggt[pl.ds(k * 16, 16)]
            aggt[pl.ds(k * 16, 16)] = jnp.where(jnp.abs(v) < jnp.inf, v, 0.0)
            return c
        lax.fori_loop(0, rng * 4, fixb, 0)

        @pl.when(w < _NW - 1)
        def _():
            pltpu.sync_copy(aggt.at[pl.ds(0, rng * 64)],
                            agg_hbm.at[pl.ds(base * 64, rng * 64)])

        @pl.when(w == _NW - 1)
        def _():
            pltpu.sync_copy(aggt.at[pl.ds(0, lastr * 64)],
                            agg_hbm.at[pl.ds(base * 64, lastr * 64)])

    return scsegmax


def _upd_body(x_ref, ag_ref, wox_ref, woa_ref, bo_ref, o_ref):
    o_ref[...] = (jnp.dot(x_ref[...], wox_ref[...], preferred_element_type=F32)
                  + jnp.dot(ag_ref[...], woa_ref[...], preferred_element_type=F32)
                  + bo_ref[...])


def _update(x, agg, wox, woa, b_out):
    n = x.shape[0]
    blk = 2000
    return pl.pallas_call(
        _upd_body,
        grid=(n // blk,),
        in_specs=[
            pl.BlockSpec((blk, 128), lambda i: (i, 0)),
            pl.BlockSpec((blk, 64), lambda i: (i, 0)),
            pl.BlockSpec((128, 64), lambda i: (0, 0)),
            pl.BlockSpec((64, 64), lambda i: (0, 0)),
            pl.BlockSpec((1, 64), lambda i: (0, 0)),
        ],
        out_specs=pl.BlockSpec((blk, 64), lambda i: (i, 0)),
        out_shape=jax.ShapeDtypeStruct((n, 64), F32),
    )(x, agg, wox, woa, b_out.reshape(1, 64))


@jax.jit
def kernel(x, adj, edge_attr, W_msg, b_msg, W1, b1, W2, b2, W_out, b_out):
    n = x.shape[0]
    e = adj.shape[1]
    src = adj[0].astype(I32)
    dst = adj[1].astype(I32)

    xsd = _xproj(x, jnp.concatenate([W_msg[:128], W_msg[128:256]], axis=1))
    eawh = _eaw(edge_attr, W_msg[256:], b_msg)

    m0h = _make_scgather(e)(src, dst, xsd)

    h2 = _mlp(m0h, eawh, W1, b1, W2, b2)

    agg = _make_scsegmax(e, n)(dst, h2).reshape(n, 64)

    return _update(x, agg, W_out[:128], W_out[128:], b_out)


# speedup vs baseline: 1.8245x; 1.0179x over previous
import functools

import jax
import jax.numpy as jnp
from jax import lax
from jax.experimental import pallas as pl
from jax.experimental.pallas import tpu as pltpu
from jax.experimental.pallas import tpu_sc as plsc

F32 = jnp.float32
I32 = jnp.int32

_NC, _NS, _NW = 2, 16, 32


def _mesh():
    return plsc.VectorSubcoreMesh(core_axis_name="c", subcore_axis_name="s")


_SC_PARAMS = pltpu.CompilerParams(needs_layout_passes=False)


def _wid():
    return lax.axis_index("s") * _NC + lax.axis_index("c")


def _xproj_body(x_ref, w_ref, o_ref):
    o_ref[...] = jnp.dot(x_ref[...], w_ref[...], preferred_element_type=F32)


def _xproj(x, wcat):
    n = x.shape[0]
    blk = 2000
    return pl.pallas_call(
        _xproj_body,
        grid=(n // blk,),
        in_specs=[
            pl.BlockSpec((blk, 128), lambda i: (i, 0)),
            pl.BlockSpec((128, 128), lambda i: (0, 0)),
        ],
        out_specs=pl.BlockSpec((blk, 128), lambda i: (i, 0)),
        out_shape=jax.ShapeDtypeStruct((n, 128), F32),
    )(x, wcat)


def _eaw_body(e1_ref, e2_ref, we_ref, bm_ref, o_ref):
    o_ref[:, :64] = (jnp.dot(e1_ref[...], we_ref[...], preferred_element_type=F32)
                     + bm_ref[...])
    o_ref[:, 64:] = (jnp.dot(e2_ref[...], we_ref[...], preferred_element_type=F32)
                     + bm_ref[...])


def _eaw(edge_attr, we, b_msg):
    e = edge_attr.shape[0]
    e2 = e // 2
    blk = 1000
    grid = e2 // blk
    return pl.pallas_call(
        _eaw_body,
        grid=(grid,),
        in_specs=[
            pl.BlockSpec((blk, 64), lambda i: (i, 0)),
            pl.BlockSpec((blk, 64), lambda i, g=grid: (i + g, 0)),
            pl.BlockSpec((64, 64), lambda i: (0, 0)),
            pl.BlockSpec((1, 64), lambda i: (0, 0)),
        ],
        out_specs=pl.BlockSpec((blk, 128), lambda i: (i, 0)),
        out_shape=jax.ShapeDtypeStruct((e2, 128), F32),
    )(edge_attr, edge_attr, we, b_msg.reshape(1, 64))


def _make_scgather(e):
    e2 = e // 2
    bc = 128
    nbc = e2 // bc
    per = nbc // _NW
    extra = nbc - per * _NW

    @functools.partial(
        pl.kernel,
        out_type=jax.ShapeDtypeStruct((e2, 128), F32),
        mesh=_mesh(),
        compiler_params=_SC_PARAMS,
        scratch_types=[
            pltpu.VMEM((bc,), I32),
            pltpu.VMEM((bc,), I32),
            pltpu.VMEM((bc,), I32),
            pltpu.VMEM((bc,), I32),
            pltpu.VMEM((bc, 128), F32),
            pltpu.VMEM((bc, 128), F32),
            pltpu.VMEM((bc, 128), F32),
            pltpu.VMEM((bc, 128), F32),
            pltpu.VMEM((bc, 128), F32),
            pltpu.SemaphoreType.DMA,
        ],
    )
    def scgather(src_hbm, dst_hbm, xsd_hbm, m0_hbm,
                 s1i, d1i, s2i, d2i, bs1, bd1, bs2, bd2, bm, sem):
        w = _wid()
        nb = jnp.where(w < extra, per + 1, per)

        def body(t, carry):
            off = pl.multiple_of((w + _NW * t) * bc, 8)
            pltpu.sync_copy(src_hbm.at[pl.ds(off, bc)], s1i)
            pltpu.sync_copy(dst_hbm.at[pl.ds(off, bc)], d1i)
            pltpu.sync_copy(src_hbm.at[pl.ds(e2 + off, bc)], s2i)
            pltpu.sync_copy(dst_hbm.at[pl.ds(e2 + off, bc)], d2i)
            cps = [
                pltpu.async_copy(xsd_hbm.at[s1i], bs1, sem),
                pltpu.async_copy(xsd_hbm.at[d1i], bd1, sem),
                pltpu.async_copy(xsd_hbm.at[s2i], bs2, sem),
                pltpu.async_copy(xsd_hbm.at[d2i], bd2, sem),
            ]
            for cp in cps:
                cp.wait()

            @functools.partial(plsc.parallel_loop, 0, bc, unroll=4)
            def row(r):
                for cc in range(4):
                    lo = pl.ds(cc * 16, 16)
                    hi = pl.ds(64 + cc * 16, 16)
                    bm[r, lo] = bs1[r, lo] + bd1[r, hi]
                    bm[r, hi] = bs2[r, lo] + bd2[r, hi]

            pltpu.sync_copy(bm, m0_hbm.at[pl.ds(off, bc)])
            return carry

        lax.fori_loop(0, nb, body, 0)

    return scgather


def _mlp_body(m0_ref, ew_ref, w1_ref, b1_ref, w2_ref, b2_ref, h_ref):
    m = m0_ref[...] + ew_ref[...]
    t = jnp.maximum(jnp.dot(m, w1_ref[...], preferred_element_type=F32)
                    + b1_ref[...], 0.0)
    h_ref[...] = jnp.dot(t, w2_ref[...], preferred_element_type=F32) + b2_ref[...]


def _mlp(m0h, eawh, w1, b1, w2, b2):
    e2 = m0h.shape[0]
    blk = 1000
    zero = jnp.zeros((64, 64), F32)
    w1d = jnp.block([[w1, zero], [zero, w1]])
    w2d = jnp.block([[w2, zero], [zero, w2]])
    b1d = jnp.concatenate([b1, b1]).reshape(1, 128)
    b2d = jnp.concatenate([b2, b2]).reshape(1, 128)
    row = lambda i: (i, 0)
    whole = lambda i: (0, 0)
    return pl.pallas_call(
        _mlp_body,
        grid=(e2 // blk,),
        in_specs=[
            pl.BlockSpec((blk, 128), row),
            pl.BlockSpec((blk, 128), row),
            pl.BlockSpec((128, 128), whole),
            pl.BlockSpec((1, 128), whole),
            pl.BlockSpec((128, 128), whole),
            pl.BlockSpec((1, 128), whole),
        ],
        out_specs=pl.BlockSpec((blk, 128), row),
        out_shape=jax.ShapeDtypeStruct((e2, 128), F32),
    )(m0h, eawh, w1d, b1d, w2d, b2d)


def _make_scsegmax(e, n):
    e2 = e // 2
    rng = (n + _NW - 1) // _NW
    lastr = n - (_NW - 1) * rng
    chunk = 8000
    nchunks = e // chunk
    cap = chunk + 224

    @functools.partial(
        pl.kernel,
        out_type=jax.ShapeDtypeStruct((n * 64,), F32),
        mesh=_mesh(),
        compiler_params=_SC_PARAMS,
        scratch_types=[
            pltpu.VMEM(((rng + 1) * 64,), F32),
            pltpu.VMEM((chunk,), I32),
            pltpu.VMEM((cap,), I32),
            pltpu.VMEM((cap,), I32),
            pltpu.VMEM((128, 128), F32),
            pltpu.VMEM((128, 128), F32),
            pltpu.SemaphoreType.DMA,
            pltpu.SemaphoreType.DMA,
        ],
    )
    def scsegmax(dst_hbm, h_hbm, agg_hbm, aggt, dbuf, capd, capg,
                 hb0, hb1, sem0, sem1):
        w = _wid()
        base = w * rng
        ninf = jnp.full((16,), -jnp.inf, F32)

        @functools.partial(plsc.parallel_loop, 0, (rng + 1) * 4, unroll=8)
        def initb(k):
            aggt[pl.ds(k * 16, 16)] = ninf

        iota16 = lax.iota(I32, 16)
        trash = chunk + 208 + iota16
        ones = jnp.full((16,), 1, I32)
        zeros16 = jnp.zeros((16,), I32)

        def do_chunk(cidx, carry):
            off = pl.multiple_of(cidx * chunk, 8)
            pltpu.sync_copy(dst_hbm.at[pl.ds(off, chunk)], dbuf)

            def scan(i, cnt_v):
                d16 = dbuf[pl.ds(i * 16, 16)]
                t = d16 - base
                m = (t >= 0) & (t < rng)
                ev = off + i * 16 + iota16
                half = ev >= e2
                vv = t + jnp.where(half, 32768, 0)
                gv = ev - jnp.where(half, e2, 0)
                cs = plsc.cumsum(jnp.where(m, ones, zeros16))
                pos = jnp.where(m, cnt_v + cs - 1, trash)
                plsc.store_scatter(capd, [pos], vv)
                plsc.store_scatter(capg, [pos], gv)
                pc = plsc.all_reduce_population_count(m)
                return cnt_v + pc

            cnt_v = plsc.parallel_loop(0, chunk // 16, unroll=4,
                                       carry=zeros16)(scan)
            cnt = cnt_v[0]

            capd[pl.ds(cnt, 16)] = jnp.full((16,), rng, I32)
            def padb(j, c):
                capg[pl.ds(cnt + j * 16, 16)] = zeros16
                return c
            lax.fori_loop(0, 8, padb, 0)

            nslab = (cnt + 127) // 128

            def start(s, hb, sem):
                sl = pl.ds(pl.multiple_of(s * 128, 8), 128)
                return pltpu.async_copy(h_hbm.at[capg.at[sl]], hb, sem)

            def accum(s, hb):
                lo = s * 128
                hi = cnt - lo

                def group(g):
                    @pl.when(g * 16 < hi)
                    def _():
                        vv16 = capd[pl.ds(lo + g * 16, 16)]
                        for k in range(16):
                            vv = vv16[k]
                            d = vv & 32767
                            hb_col = (vv >> 15) * 64
                            rb = pl.multiple_of(d * 64, 8)
                            j = g * 16 + k
                            for cc in range(4):
                                hv = hb[j, pl.ds(hb_col + cc * 16, 16)]
                                av = aggt[pl.ds(rb + cc * 16, 16)]
                                aggt[pl.ds(rb + cc * 16, 16)] = jnp.maximum(av, hv)
                for g in range(8):
                    group(g)

            cp0 = start(0, hb0, sem0)

            def slab(s, c):
                even = lax.rem(s, 2) == 0

                @pl.when(even)
                def _():
                    pltpu.make_async_copy(h_hbm.at[capg.at[pl.ds(0, 128)]], hb0, sem0).wait()
                    @pl.when(s + 1 < nslab)
                    def _():
                        start(s + 1, hb1, sem1)
                    accum(s, hb0)

                @pl.when(jnp.logical_not(even))
                def _():
                    pltpu.make_async_copy(h_hbm.at[capg.at[pl.ds(0, 128)]], hb1, sem1).wait()
                    @pl.when(s + 1 < nslab)
                    def _():
                        start(s + 1, hb0, sem0)
                    accum(s, hb1)

                return c

            lax.fori_loop(0, nslab, slab, 0)
            return carry

        lax.fori_loop(0, nchunks, do_chunk, 0)

        @functools.partial(plsc.parallel_loop, 0, rng * 4, unroll=8)
        def fixb(k):
            v = aggt[pl.ds(k * 16, 16)]
            aggt[pl.ds(k * 16, 16)] = jnp.where(jnp.abs(v) < jnp.inf, v, 0.0)

        @pl.when(w < _NW - 1)
        def _():
            pltpu.sync_copy(aggt.at[pl.ds(0, rng * 64)],
                            agg_hbm.at[pl.ds(base * 64, rng * 64)])

        @pl.when(w == _NW - 1)
        def _():
            pltpu.sync_copy(aggt.at[pl.ds(0, lastr * 64)],
                            agg_hbm.at[pl.ds(base * 64, lastr * 64)])

    return scsegmax


def _upd_body(x_ref, ag_ref, wox_ref, woa_ref, bo_ref, o_ref):
    o_ref[...] = (jnp.dot(x_ref[...], wox_ref[...], preferred_element_type=F32)
                  + jnp.dot(ag_ref[...], woa_ref[...], preferred_element_type=F32)
                  + bo_ref[...])


def _update(x, agg, wox, woa, b_out):
    n = x.shape[0]
    blk = 2000
    return pl.pallas_call(
        _upd_body,
        grid=(n // blk,),
        in_specs=[
            pl.BlockSpec((blk, 128), lambda i: (i, 0)),
            pl.BlockSpec((blk, 64), lambda i: (i, 0)),
            pl.BlockSpec((128, 64), lambda i: (0, 0)),
            pl.BlockSpec((64, 64), lambda i: (0, 0)),
            pl.BlockSpec((1, 64), lambda i: (0, 0)),
        ],
        out_specs=pl.BlockSpec((blk, 64), lambda i: (i, 0)),
        out_shape=jax.ShapeDtypeStruct((n, 64), F32),
    )(x, agg, wox, woa, b_out.reshape(1, 64))


@jax.jit
def kernel(x, adj, edge_attr, W_msg, b_msg, W1, b1, W2, b2, W_out, b_out):
    n = x.shape[0]
    e = adj.shape[1]
    src = adj[0].astype(I32)
    dst = adj[1].astype(I32)

    xsd = _xproj(x, jnp.concatenate([W_msg[:128], W_msg[128:256]], axis=1))
    eawh = _eaw(edge_attr, W_msg[256:], b_msg)

    m0h = _make_scgather(e)(src, dst, xsd)

    h2 = _mlp(m0h, eawh, W1, b1, W2, b2)

    agg = _make_scsegmax(e, n)(dst, h2).reshape(n, 64)

    return _update(x, agg, W_out[:128], W_out[128:], b_out)
